# baseline (speedup 1.0000x reference)
import functools

import jax
import jax.numpy as jnp
from jax import lax
from jax.experimental import pallas as pl
from jax.experimental.pallas import tpu as pltpu
from jax.experimental.pallas import tpu_sc as plsc

D = 128
NC = 2
NS = 16
NW = NC * NS
CH = 80
IG = 8
NB = 4
N_ACC = 10240
RPT = N_ACC // NS


def _sc_agg(sup, src3, dst3, zeros, K):
    mesh = plsc.VectorSubcoreMesh(core_axis_name="c", subcore_axis_name="s")
    NG = K // IG

    @functools.partial(
        pl.kernel,
        out_type=jax.ShapeDtypeStruct((NC * N_ACC, D), jnp.float32),
        mesh=mesh,
        scratch_types=[
            pltpu.VMEM((2, IG, CH), jnp.int32),
            pltpu.VMEM((2, IG, CH), jnp.int32),
            pltpu.VMEM((NB, CH, D), jnp.float32),
            pltpu.VMEM_SHARED((N_ACC, D), jnp.float32),
            [pltpu.SemaphoreType.DMA] * NB,
            [pltpu.SemaphoreType.DMA] * NB,
            pltpu.SemaphoreType.DMA,
            pltpu.SemaphoreType.DMA,
        ],
    )
    def k(sup_hbm, src_hbm, dst_hbm, zero_hbm, out_hbm,
          sidx, didx, rows, acc, gsems, ssems, isem0, isem1):
        cid = lax.axis_index("c")
        sid = lax.axis_index("s")
        wid = sid * NC + cid
        isems = (isem0, isem1)

        def idx_start(g, p):
            pltpu.async_copy(src_hbm.at[wid, pl.ds(g * IG, IG)], sidx.at[p],
                             isems[p])
            pltpu.async_copy(dst_hbm.at[wid, pl.ds(g * IG, IG)], didx.at[p],
                             isems[p])

        def idx_wait(p):
            pltpu.make_async_copy(src_hbm.at[wid, pl.ds(0, IG)], sidx.at[p],
                                  isems[p]).wait()
            pltpu.make_async_copy(dst_hbm.at[wid, pl.ds(0, IG)], didx.at[p],
                                  isems[p]).wait()

        def gather_start(p, j_local, b):
            pltpu.async_copy(sup_hbm.at[sidx.at[p, j_local]], rows.at[b],
                             gsems[b])

        def gather_wait(p, j_local, b):
            pltpu.make_async_copy(sup_hbm.at[sidx.at[p, j_local]], rows.at[b],
                                  gsems[b]).wait()

        def scatter_start(p, j_local, b):
            pltpu.async_copy(rows.at[b], acc.at[didx.at[p, j_local]],
                             ssems[b], add=True)

        def scatter_wait(p, j_local, b):
            pltpu.make_async_copy(rows.at[b], acc.at[didx.at[p, j_local]],
                                  ssems[b]).wait()

        idx_start(0, 0)
        pltpu.sync_copy(zero_hbm.at[wid], acc.at[pl.ds(sid * RPT, RPT)])
        idx_wait(0)
        plsc.subcore_barrier()
        gather_start(0, 0, 0)
        gather_start(0, 1, 1)

        def group(g, p):
            for j in range(IG):
                b = j % NB
                gather_wait(p, j, b)
                scatter_start(p, j, b)
                bt = (j + 2) % NB
                if j < 2:
                    @pl.when(g > 0)
                    def _():
                        scatter_wait(p ^ 1, j + IG - 2, bt)
                else:
                    scatter_wait(p, j - 2, bt)
                if j < IG - 2:
                    gather_start(p, j + 2, bt)
                else:
                    @pl.when(g + 1 < NG)
                    def _():
                        gather_start(p ^ 1, j + 2 - IG, bt)
                if j == 1:
                    @pl.when(g + 1 < NG)
                    def _():
                        idx_start(g + 1, p ^ 1)
                if j == IG - 2:
                    @pl.when(g + 1 < NG)
                    def _():
                        idx_wait(p ^ 1)

        def pair(t, carry):
            group(t * 2, 0)
            group(t * 2 + 1, 1)
            return carry

        lax.fori_loop(0, NG // 2, pair, 0)
        scatter_wait(1, IG - 2, (IG - 2) % NB)
        scatter_wait(1, IG - 1, (IG - 1) % NB)
        plsc.subcore_barrier()
        pltpu.sync_copy(
            acc.at[pl.ds(sid * RPT, RPT)],
            out_hbm.at[pl.ds(cid * N_ACC + sid * RPT, RPT)],
        )

    return k(sup, src3, dst3, zeros)


def _tc_layer(P, W, b2d, n_rows):
    BR = 2000
    grid = n_rows // BR
    P3 = P.reshape(NC, N_ACC, D)

    def body(p0_ref, p1_ref, w_ref, b_ref, o_ref):
        agg = p0_ref[0] + p1_ref[0]
        h = jnp.dot(agg, w_ref[...], preferred_element_type=jnp.float32)
        o_ref[...] = jnp.maximum(h + b_ref[...], 0.0)

    return pl.pallas_call(
        body,
        grid=(grid,),
        in_specs=[
            pl.BlockSpec((1, BR, D), lambda i: (0, i, 0)),
            pl.BlockSpec((1, BR, D), lambda i: (1, i, 0)),
            pl.BlockSpec((D, D), lambda i: (0, 0)),
            pl.BlockSpec((1, D), lambda i: (0, 0)),
        ],
        out_specs=pl.BlockSpec((BR, D), lambda i: (i, 0)),
        out_shape=jax.ShapeDtypeStruct((n_rows, D), jnp.float32),
    )(P3, P3, W, b2d)


def kernel(x, edges, W1, b1, W2, b2):
    if x.ndim == 3:
        x = jnp.squeeze(x)
    n = x.shape[0]
    e = edges.shape[1]
    per_blk = NW * CH * IG * 2
    k_chunks = (-(-e // per_blk)) * IG * 2
    e_pad = NW * k_chunks * CH
    pad_iota = jnp.arange(e_pad - e, dtype=jnp.int32)
    src = jnp.concatenate(
        [edges[0], pad_iota % n]).reshape(NW, k_chunks, CH)
    dst = jnp.concatenate(
        [edges[1], n + pad_iota % (N_ACC - n)]).reshape(NW, k_chunks, CH)
    zeros = jnp.zeros((NW, RPT, D), jnp.float32)
    b1_2d = b1.reshape(1, D)
    b2_2d = b2.reshape(1, D)

    P1 = _sc_agg(x, src, dst, zeros, k_chunks)
    x1 = _tc_layer(P1, W1, b1_2d, n)
    P2 = _sc_agg(x1, src, dst, zeros, k_chunks)
    return _tc_layer(P2, W2, b2_2d, n)


# speedup vs baseline: 1.0958x; 1.0958x over previous
import functools

import jax
import jax.numpy as jnp
from jax import lax
from jax.experimental import pallas as pl
from jax.experimental.pallas import tpu as pltpu
from jax.experimental.pallas import tpu_sc as plsc

D = 128
NC = 2
NS = 16
NW = NC * NS
CH = 128
IG = 8
N_ACC = 10240
RPT = N_ACC // NS


def _sc_agg(sup, src3, dst3, zeros, K):
    mesh = plsc.VectorSubcoreMesh(core_axis_name="c", subcore_axis_name="s")
    NG = K // IG

    @functools.partial(
        pl.kernel,
        out_type=jax.ShapeDtypeStruct((NC * N_ACC, D), jnp.float32),
        mesh=mesh,
        scratch_types=[
            pltpu.VMEM((2, IG, CH), jnp.int32),
            pltpu.VMEM((2, IG, CH), jnp.int32),
            pltpu.VMEM((CH, D), jnp.float32),
            pltpu.VMEM((CH, D), jnp.float32),
            pltpu.VMEM_SHARED((N_ACC, D), jnp.float32),
            pltpu.SemaphoreType.DMA,
            pltpu.SemaphoreType.DMA,
            pltpu.SemaphoreType.DMA,
            pltpu.SemaphoreType.DMA,
        ],
    )
    def k(sup_hbm, src_hbm, dst_hbm, zero_hbm, out_hbm,
          sidx, didx, rows0, rows1, acc, sem0, sem1, isem0, isem1):
        cid = lax.axis_index("c")
        sid = lax.axis_index("s")
        wid = sid * NC + cid
        isems = (isem0, isem1)
        bufs = ((rows0, sem0), (rows1, sem1))

        def idx_start(g, p):
            pltpu.async_copy(src_hbm.at[wid, pl.ds(g * IG, IG)], sidx.at[p],
                             isems[p])
            pltpu.async_copy(dst_hbm.at[wid, pl.ds(g * IG, IG)], didx.at[p],
                             isems[p])

        def idx_wait(p):
            pltpu.make_async_copy(src_hbm.at[wid, pl.ds(0, IG)], sidx.at[p],
                                  isems[p]).wait()
            pltpu.make_async_copy(dst_hbm.at[wid, pl.ds(0, IG)], didx.at[p],
                                  isems[p]).wait()

        def gather_start(p, j_local, rbuf, sem):
            pltpu.async_copy(sup_hbm.at[sidx.at[p, j_local]], rbuf, sem)

        def gather_wait(p, j_local, rbuf, sem):
            pltpu.make_async_copy(sup_hbm.at[sidx.at[p, j_local]], rbuf,
                                  sem).wait()

        idx_start(0, 0)
        idx_wait(0)
        idx_start(1, 1)
        gather_start(0, 0, rows0, sem0)
        gather_start(0, 1, rows1, sem1)
        pltpu.sync_copy(zero_hbm.at[wid], acc.at[pl.ds(sid * RPT, RPT)])
        plsc.subcore_barrier()

        def group(g, p):
            for j_local in range(IG):
                rbuf, sem = bufs[j_local % 2]
                gather_wait(p, j_local, rbuf, sem)
                pltpu.sync_copy(rbuf, acc.at[didx.at[p, j_local]], add=True)
                if j_local == IG - 2:
                    @pl.when(g + 1 < NG)
                    def _():
                        idx_wait(p ^ 1)
                if j_local + 2 < IG:
                    gather_start(p, j_local + 2, rbuf, sem)
                else:
                    nxt = j_local + 2 - IG

                    @pl.when(g + 1 < NG)
                    def _():
                        gather_start(p ^ 1, nxt, rbuf, sem)
                if j_local == IG - 1:
                    @pl.when(g + 2 < NG)
                    def _():
                        idx_start(g + 2, p)

        def pair(t, carry):
            group(t * 2, 0)
            group(t * 2 + 1, 1)
            return carry

        lax.fori_loop(0, NG // 2, pair, 0)
        plsc.subcore_barrier()
        pltpu.sync_copy(
            acc.at[pl.ds(sid * RPT, RPT)],
            out_hbm.at[pl.ds(cid * N_ACC + sid * RPT, RPT)],
        )

    return k(sup, src3, dst3, zeros)


def _tc_layer(P, W, b2d, n_rows):
    BR = 2000
    grid = n_rows // BR
    P3 = P.reshape(NC, N_ACC, D)

    def body(p0_ref, p1_ref, w_ref, b_ref, o_ref):
        agg = p0_ref[0] + p1_ref[0]
        h = jnp.dot(agg, w_ref[...], preferred_element_type=jnp.float32)
        o_ref[...] = jnp.maximum(h + b_ref[...], 0.0)

    return pl.pallas_call(
        body,
        grid=(grid,),
        in_specs=[
            pl.BlockSpec((1, BR, D), lambda i: (0, i, 0)),
            pl.BlockSpec((1, BR, D), lambda i: (1, i, 0)),
            pl.BlockSpec((D, D), lambda i: (0, 0)),
            pl.BlockSpec((1, D), lambda i: (0, 0)),
        ],
        out_specs=pl.BlockSpec((BR, D), lambda i: (i, 0)),
        out_shape=jax.ShapeDtypeStruct((n_rows, D), jnp.float32),
    )(P3, P3, W, b2d)


def kernel(x, edges, W1, b1, W2, b2):
    if x.ndim == 3:
        x = jnp.squeeze(x)
    n = x.shape[0]
    e = edges.shape[1]
    per_blk = NW * CH * IG * 2
    k_chunks = (-(-e // per_blk)) * IG * 2
    e_pad = NW * k_chunks * CH
    pad_iota = jnp.arange(e_pad - e, dtype=jnp.int32)
    src = jnp.concatenate(
        [edges[0], pad_iota % n]).reshape(NW, k_chunks, CH)
    dst = jnp.concatenate(
        [edges[1], n + pad_iota % (N_ACC - n)]).reshape(NW, k_chunks, CH)
    zeros = jnp.zeros((NW, RPT, D), jnp.float32)
    b1_2d = b1.reshape(1, D)
    b2_2d = b2.reshape(1, D)

    P1 = _sc_agg(x, src, dst, zeros, k_chunks)
    x1 = _tc_layer(P1, W1, b1_2d, n)
    P2 = _sc_agg(x1, src, dst, zeros, k_chunks)
    return _tc_layer(P2, W2, b2_2d, n)
